# Initial kernel scaffold; baseline (speedup 1.0000x reference)
import jax
import jax.numpy as jnp
from jax.experimental import pallas as pl


def kernel(x, edge_index, types, W_msg, W_cells, U_cells, b_cells):
    raise NotImplementedError("write your pallas kernel here")


# speedup vs baseline: 3.0222x; 3.0222x over previous
import functools

import jax
import jax.numpy as jnp
from jax import lax
from jax.experimental import pallas as pl
from jax.experimental.pallas import tpu as pltpu
from jax.experimental.pallas import tpu_sc as plsc


def _msg_matmul(x, W_msg, nc, hh, ba):
    n, d = x.shape
    nb = n // ba

    def body(x_ref, w_ref, o_ref):
        o_ref[...] = jnp.dot(x_ref[...], w_ref[...],
                             preferred_element_type=jnp.float32)

    return pl.pallas_call(
        body,
        grid=(nc, nb),
        in_specs=[
            pl.BlockSpec((ba, d), lambda c, i: (i, 0)),
            pl.BlockSpec((d, hh), lambda c, i: (0, c)),
        ],
        out_specs=pl.BlockSpec((ba, hh), lambda c, i: (c * nb + i, 0)),
        out_shape=jax.ShapeDtypeStruct((nc * n, hh), jnp.float32),
    )(x, W_msg)


def _segment_sum_sc(src2, dst2, y2, zeros, n, hh, nc, ns, ch):
    npc = dst2.shape[0]
    cpt = npc // ns
    n_pad = zeros.shape[0] * ns
    rpt = zeros.shape[0]

    mesh = plsc.VectorSubcoreMesh(core_axis_name="c", subcore_axis_name="s")

    @functools.partial(
        pl.kernel,
        out_type=jax.ShapeDtypeStruct((nc * n_pad, hh), jnp.float32),
        mesh=mesh,
        scratch_types=[
            pltpu.VMEM((cpt, ch), jnp.int32),
            pltpu.VMEM((cpt, ch), jnp.int32),
            pltpu.VMEM((ch, hh), jnp.float32),
            pltpu.VMEM_SHARED((n_pad, hh), jnp.float32),
            pltpu.SemaphoreType.DMA,
        ],
    )
    def segsum(src_hbm, dst_hbm, y_hbm, z_hbm, out_hbm, gi, si, rows, acc, sem):
        c = lax.axis_index("c")
        s = lax.axis_index("s")
        pltpu.sync_copy(z_hbm, acc.at[pl.ds(s * rpt, rpt)])
        pltpu.sync_copy(src_hbm.at[pl.ds(c * npc + s * cpt, cpt)], gi)
        pltpu.sync_copy(dst_hbm.at[pl.ds(s * cpt, cpt)], si)
        plsc.subcore_barrier()

        def body(j, carry):
            pltpu.async_copy(y_hbm.at[gi.at[j]], rows, sem).wait()
            pltpu.sync_copy(rows, acc.at[si.at[j]], add=True)
            return carry

        lax.fori_loop(0, cpt, body, 0)
        plsc.subcore_barrier()
        pltpu.sync_copy(acc.at[pl.ds(s * rpt, rpt)],
                        out_hbm.at[pl.ds(c * n_pad + s * rpt, rpt)])

    return segsum(src2, dst2, y2, zeros)


def _apply_cells(agg2, x, types2, W_cells, U_cells, b_cells, hh, bc):
    n, d = x.shape
    t1, h, _ = W_cells.shape
    nb = n // bc

    def body(agg_ref, x_ref, t_ref, w_ref, u_ref, b_ref, o_ref):
        a0 = agg_ref[0]
        a1 = agg_ref[1]
        xv = x_ref[...]
        tv = t_ref[...]
        out = jnp.zeros((bc, h), jnp.float32)
        for k in range(t1):
            wk = w_ref[k]
            pre = (jnp.dot(a0, wk[:hh, :], preferred_element_type=jnp.float32)
                   + jnp.dot(a1, wk[hh:, :], preferred_element_type=jnp.float32)
                   + jnp.dot(xv, u_ref[k], preferred_element_type=jnp.float32)
                   + b_ref[k])
            out = out + jnp.where(tv == k, jnp.tanh(pre), 0.0)
        o_ref[...] = out

    return pl.pallas_call(
        body,
        grid=(nb,),
        in_specs=[
            pl.BlockSpec((2, bc, hh), lambda i: (0, i, 0)),
            pl.BlockSpec((bc, d), lambda i: (i, 0)),
            pl.BlockSpec((bc, 1), lambda i: (i, 0)),
            pl.BlockSpec((t1, h, h), lambda i: (0, 0, 0)),
            pl.BlockSpec((t1, d, h), lambda i: (0, 0, 0)),
            pl.BlockSpec((t1, h), lambda i: (0, 0)),
        ],
        out_specs=pl.BlockSpec((bc, h), lambda i: (i, 0)),
        out_shape=jax.ShapeDtypeStruct((n, h), jnp.float32),
    )(agg2, x, types2, W_cells, U_cells, b_cells)


def kernel(x, edge_index, types, W_msg, W_cells, U_cells, b_cells):
    n, d = x.shape
    h = W_msg.shape[1]
    e = edge_index.shape[1]
    hh = h // 2
    nc, ns, ch = 2, 16, 128

    gran = ns * ch * 8
    e_pad = ((e + gran - 1) // gran) * gran
    pad = e_pad - e
    src = edge_index[0]
    dst = edge_index[1]
    src_p = jnp.concatenate([src, jnp.zeros((pad,), jnp.int32)])
    dst_p = jnp.concatenate([dst, jnp.full((pad,), n, jnp.int32)])
    src2 = jnp.stack([src_p, src_p + n]).reshape(nc * (e_pad // ch), ch)
    dst2 = dst_p.reshape(e_pad // ch, ch)

    n_pad = ((n + 1 + ns * 8 - 1) // (ns * 8)) * (ns * 8)
    zeros = jnp.zeros((n_pad // ns, hh), jnp.float32)

    y2 = _msg_matmul(x, W_msg, nc, hh, ba=1000)
    agg_flat = _segment_sum_sc(src2, dst2, y2, zeros, n, hh, nc, ns, ch)
    agg2 = agg_flat.reshape(nc, n_pad, hh)
    out = _apply_cells(agg2, x, types.reshape(n, 1),
                       W_cells, U_cells, b_cells, hh, bc=400)
    return out


# speedup vs baseline: 6.9766x; 2.3084x over previous
import functools

import jax
import jax.numpy as jnp
from jax import lax
from jax.experimental import pallas as pl
from jax.experimental.pallas import tpu as pltpu
from jax.experimental.pallas import tpu_sc as plsc


def _msg_matmul(x, W_msg, nc, hh, ba):
    n, d = x.shape
    nb = n // ba

    def body(x_ref, w_ref, o_ref):
        o_ref[...] = jnp.dot(x_ref[...], w_ref[...],
                             preferred_element_type=jnp.float32)

    return pl.pallas_call(
        body,
        grid=(nc, nb),
        in_specs=[
            pl.BlockSpec((ba, d), lambda c, i: (i, 0)),
            pl.BlockSpec((d, hh), lambda c, i: (0, c)),
        ],
        out_specs=pl.BlockSpec((ba, hh), lambda c, i: (c * nb + i, 0)),
        out_shape=jax.ShapeDtypeStruct((nc * n, hh), jnp.float32),
    )(x, W_msg)


def _segment_sum_sc(src2, dst2, y2, zeros, n, hh, nc, ns, ch):
    npc = dst2.shape[0]
    cpt = npc // ns
    n_pad = zeros.shape[0] * ns
    rpt = zeros.shape[0]

    mesh = plsc.VectorSubcoreMesh(core_axis_name="c", subcore_axis_name="s")

    gblk = 16
    nblk = cpt // gblk
    assert cpt % gblk == 0

    @functools.partial(
        pl.kernel,
        out_type=jax.ShapeDtypeStruct((nc * n_pad, hh), jnp.float32),
        mesh=mesh,
        scratch_types=[
            pltpu.VMEM((cpt, ch), jnp.int32),
            pltpu.VMEM((2, gblk, ch), jnp.int32),
            pltpu.VMEM((2, ch, hh), jnp.float32),
            pltpu.VMEM_SHARED((n_pad, hh), jnp.float32),
            pltpu.SemaphoreType.DMA,
            pltpu.SemaphoreType.DMA,
            pltpu.SemaphoreType.DMA,
        ],
    )
    def segsum(src_hbm, dst_hbm, y_hbm, z_hbm, out_hbm, gi, si2, rows, acc,
               gs0, gs1, ssem):
        c = lax.axis_index("c")
        s = lax.axis_index("s")
        gsem = (gs0, gs1)
        pltpu.sync_copy(z_hbm, acc.at[pl.ds(s * rpt, rpt)])
        pltpu.sync_copy(src_hbm.at[pl.ds(c * npc + s * cpt, cpt)], gi)
        dbase = s * cpt
        pltpu.sync_copy(dst_hbm.at[pl.ds(dbase, gblk)], si2.at[0])
        plsc.subcore_barrier()
        pltpu.async_copy(y_hbm.at[gi.at[0]], rows.at[0], gsem[0])

        def body(jo, carry):
            sl = lax.rem(jo, 2)

            @pl.when(jo > 0)
            def _():
                pltpu.make_async_copy(
                    dst_hbm.at[pl.ds(dbase + jo * gblk, gblk)],
                    si2.at[sl], ssem).wait()

            @pl.when(jo + 1 < nblk)
            def _():
                pltpu.async_copy(
                    dst_hbm.at[pl.ds(dbase + (jo + 1) * gblk, gblk)],
                    si2.at[1 - sl], ssem)

            for b in range(gblk):
                j = jo * gblk + b
                buf = b % 2

                @pl.when(j + 1 < cpt)
                def _():
                    pltpu.async_copy(y_hbm.at[gi.at[j + 1]],
                                     rows.at[1 - buf], gsem[1 - buf])

                pltpu.make_async_copy(y_hbm.at[gi.at[j]], rows.at[buf],
                                      gsem[buf]).wait()
                pltpu.sync_copy(rows.at[buf], acc.at[si2.at[sl, b]],
                                add=True)
            return carry

        lax.fori_loop(0, nblk, body, 0)
        plsc.subcore_barrier()
        pltpu.sync_copy(acc.at[pl.ds(s * rpt, rpt)],
                        out_hbm.at[pl.ds(c * n_pad + s * rpt, rpt)])

    return segsum(src2, dst2, y2, zeros)


def _apply_cells(agg2, x, types2, W_cells, U_cells, b_cells, hh, bc):
    n, d = x.shape
    t1, h, _ = W_cells.shape
    nb = n // bc

    def body(agg_ref, x_ref, t_ref, w_ref, u_ref, b_ref, o_ref):
        a0 = agg_ref[0]
        a1 = agg_ref[1]
        xv = x_ref[...]
        tv = t_ref[...]
        out = jnp.zeros((bc, h), jnp.float32)
        for k in range(t1):
            wk = w_ref[k]
            pre = (jnp.dot(a0, wk[:hh, :], preferred_element_type=jnp.float32)
                   + jnp.dot(a1, wk[hh:, :], preferred_element_type=jnp.float32)
                   + jnp.dot(xv, u_ref[k], preferred_element_type=jnp.float32)
                   + b_ref[k])
            out = out + jnp.where(tv == k, jnp.tanh(pre), 0.0)
        o_ref[...] = out

    return pl.pallas_call(
        body,
        grid=(nb,),
        in_specs=[
            pl.BlockSpec((2, bc, hh), lambda i: (0, i, 0)),
            pl.BlockSpec((bc, d), lambda i: (i, 0)),
            pl.BlockSpec((bc, 1), lambda i: (i, 0)),
            pl.BlockSpec((t1, h, h), lambda i: (0, 0, 0)),
            pl.BlockSpec((t1, d, h), lambda i: (0, 0, 0)),
            pl.BlockSpec((t1, h), lambda i: (0, 0)),
        ],
        out_specs=pl.BlockSpec((bc, h), lambda i: (i, 0)),
        out_shape=jax.ShapeDtypeStruct((n, h), jnp.float32),
    )(agg2, x, types2, W_cells, U_cells, b_cells)


def kernel(x, edge_index, types, W_msg, W_cells, U_cells, b_cells):
    n, d = x.shape
    h = W_msg.shape[1]
    e = edge_index.shape[1]
    hh = h // 2
    nc, ns, ch = 2, 16, 128

    gran = ns * ch * 8
    e_pad = ((e + gran - 1) // gran) * gran
    pad = e_pad - e
    src = edge_index[0]
    dst = edge_index[1]
    n_pad = ((n + 1 + ns * 8 - 1) // (ns * 8)) * (ns * 8)
    pad_ar = jnp.arange(pad, dtype=jnp.int32)
    src_p = jnp.concatenate([src, (pad_ar * 97) % n])
    dst_p = jnp.concatenate([dst, n + pad_ar % (n_pad - n)])
    src2 = jnp.stack([src_p, src_p + n]).reshape(nc * (e_pad // ch), ch)
    dst2 = dst_p.reshape(e_pad // ch, ch)
    zeros = jnp.zeros((n_pad // ns, hh), jnp.float32)

    y2 = _msg_matmul(x, W_msg, nc, hh, ba=1000)
    agg_flat = _segment_sum_sc(src2, dst2, y2, zeros, n, hh, nc, ns, ch)
    agg2 = agg_flat.reshape(nc, n_pad, hh)
    out = _apply_cells(agg2, x, types.reshape(n, 1),
                       W_cells, U_cells, b_cells, hh, bc=400)
    return out


# speedup vs baseline: 7.0420x; 1.0094x over previous
import functools

import jax
import jax.numpy as jnp
from jax import lax
from jax.experimental import pallas as pl
from jax.experimental.pallas import tpu as pltpu
from jax.experimental.pallas import tpu_sc as plsc


def _msg_matmul(x, W_msg, nc, hh, ba):
    n, d = x.shape
    nb = n // ba

    def body(x_ref, w_ref, o_ref):
        o_ref[...] = jnp.dot(x_ref[...], w_ref[...],
                             preferred_element_type=jnp.float32)

    return pl.pallas_call(
        body,
        grid=(nc, nb),
        in_specs=[
            pl.BlockSpec((ba, d), lambda c, i: (i, 0)),
            pl.BlockSpec((d, hh), lambda c, i: (0, c)),
        ],
        out_specs=pl.BlockSpec((ba, hh), lambda c, i: (c * nb + i, 0)),
        out_shape=jax.ShapeDtypeStruct((nc * n, hh), jnp.float32),
    )(x, W_msg)


def _segment_sum_sc(src2, dst2, y2, zeros, n, hh, nc, ns, ch):
    npc = dst2.shape[0]
    cpt = npc // ns
    n_pad = zeros.shape[0] * ns
    rpt = zeros.shape[0]

    mesh = plsc.VectorSubcoreMesh(core_axis_name="c", subcore_axis_name="s")

    gblk = 16
    nblk = cpt // gblk
    assert cpt % gblk == 0

    @functools.partial(
        pl.kernel,
        out_type=jax.ShapeDtypeStruct((nc * n_pad, hh), jnp.float32),
        mesh=mesh,
        scratch_types=[
            pltpu.VMEM((cpt, ch), jnp.int32),
            pltpu.VMEM((2, gblk, ch), jnp.int32),
            pltpu.VMEM((2, ch, hh), jnp.float32),
            pltpu.VMEM_SHARED((n_pad, hh), jnp.float32),
            pltpu.SemaphoreType.DMA,
            pltpu.SemaphoreType.DMA,
            pltpu.SemaphoreType.DMA,
            pltpu.SemaphoreType.DMA,
            pltpu.SemaphoreType.DMA,
        ],
    )
    def segsum(src_hbm, dst_hbm, y_hbm, z_hbm, out_hbm, gi, si2, rows, acc,
               gs0, gs1, ss0, ss1, isem):
        c = lax.axis_index("c")
        s = lax.axis_index("s")
        gsem = (gs0, gs1)
        ssem = (ss0, ss1)
        pltpu.sync_copy(src_hbm.at[pl.ds(c * npc + s * cpt, cpt)], gi)
        dbase = s * cpt
        pltpu.sync_copy(dst_hbm.at[pl.ds(dbase, gblk)], si2.at[0])
        pltpu.async_copy(y_hbm.at[gi.at[0]], rows.at[0], gsem[0])
        pltpu.sync_copy(z_hbm, acc.at[pl.ds(s * rpt, rpt)])
        plsc.subcore_barrier()

        def body(jo, carry):
            sl = lax.rem(jo, 2)

            @pl.when(jo > 0)
            def _():
                pltpu.make_async_copy(
                    dst_hbm.at[pl.ds(dbase + jo * gblk, gblk)],
                    si2.at[sl], isem).wait()

            @pl.when(jo + 1 < nblk)
            def _():
                pltpu.async_copy(
                    dst_hbm.at[pl.ds(dbase + (jo + 1) * gblk, gblk)],
                    si2.at[1 - sl], isem)

            prev_scatter = [None]
            for b in range(gblk):
                j = jo * gblk + b
                buf = b % 2

                if prev_scatter[0] is not None:
                    prev_scatter[0].wait()
                    prev_scatter[0] = None

                @pl.when(j + 1 < cpt)
                def _():
                    pltpu.async_copy(y_hbm.at[gi.at[j + 1]],
                                     rows.at[1 - buf], gsem[1 - buf])

                pltpu.make_async_copy(y_hbm.at[gi.at[j]], rows.at[buf],
                                      gsem[buf]).wait()
                if b < gblk - 1:
                    prev_scatter[0] = pltpu.async_copy(
                        rows.at[buf], acc.at[si2.at[sl, b]],
                        ssem[buf], add=True)
                else:
                    pltpu.sync_copy(rows.at[buf], acc.at[si2.at[sl, b]],
                                    add=True)
            return carry

        lax.fori_loop(0, nblk, body, 0)
        plsc.subcore_barrier()
        pltpu.sync_copy(acc.at[pl.ds(s * rpt, rpt)],
                        out_hbm.at[pl.ds(c * n_pad + s * rpt, rpt)])

    return segsum(src2, dst2, y2, zeros)


def _apply_cells(agg2, x, types2, W_cells, U_cells, b_cells, hh, bc):
    n, d = x.shape
    t1, h, _ = W_cells.shape
    nb = n // bc

    def body(agg_ref, x_ref, t_ref, w_ref, u_ref, b_ref, o_ref):
        a0 = agg_ref[0]
        a1 = agg_ref[1]
        xv = x_ref[...]
        tv = t_ref[...]
        out = jnp.zeros((bc, h), jnp.float32)
        for k in range(t1):
            wk = w_ref[k]
            pre = (jnp.dot(a0, wk[:hh, :], preferred_element_type=jnp.float32)
                   + jnp.dot(a1, wk[hh:, :], preferred_element_type=jnp.float32)
                   + jnp.dot(xv, u_ref[k], preferred_element_type=jnp.float32)
                   + b_ref[k])
            out = out + jnp.where(tv == k, jnp.tanh(pre), 0.0)
        o_ref[...] = out

    return pl.pallas_call(
        body,
        grid=(nb,),
        in_specs=[
            pl.BlockSpec((2, bc, hh), lambda i: (0, i, 0)),
            pl.BlockSpec((bc, d), lambda i: (i, 0)),
            pl.BlockSpec((bc, 1), lambda i: (i, 0)),
            pl.BlockSpec((t1, h, h), lambda i: (0, 0, 0)),
            pl.BlockSpec((t1, d, h), lambda i: (0, 0, 0)),
            pl.BlockSpec((t1, h), lambda i: (0, 0)),
        ],
        out_specs=pl.BlockSpec((bc, h), lambda i: (i, 0)),
        out_shape=jax.ShapeDtypeStruct((n, h), jnp.float32),
    )(agg2, x, types2, W_cells, U_cells, b_cells)


def kernel(x, edge_index, types, W_msg, W_cells, U_cells, b_cells):
    n, d = x.shape
    h = W_msg.shape[1]
    e = edge_index.shape[1]
    hh = h // 2
    nc, ns, ch = 2, 16, 128

    gran = ns * ch * 8
    e_pad = ((e + gran - 1) // gran) * gran
    pad = e_pad - e
    src = edge_index[0]
    dst = edge_index[1]
    n_pad = ((n + 1 + ns * 8 - 1) // (ns * 8)) * (ns * 8)
    pad_ar = jnp.arange(pad, dtype=jnp.int32)
    src_p = jnp.concatenate([src, (pad_ar * 97) % n])
    dst_p = jnp.concatenate([dst, n + pad_ar % (n_pad - n)])
    src2 = jnp.stack([src_p, src_p + n]).reshape(nc * (e_pad // ch), ch)
    dst2 = dst_p.reshape(e_pad // ch, ch)
    zeros = jnp.zeros((n_pad // ns, hh), jnp.float32)

    y2 = _msg_matmul(x, W_msg, nc, hh, ba=1000)
    agg_flat = _segment_sum_sc(src2, dst2, y2, zeros, n, hh, nc, ns, ch)
    agg2 = agg_flat.reshape(nc, n_pad, hh)
    out = _apply_cells(agg2, x, types.reshape(n, 1),
                       W_cells, U_cells, b_cells, hh, bc=400)
    return out


# speedup vs baseline: 7.1329x; 1.0129x over previous
import functools

import jax
import jax.numpy as jnp
from jax import lax
from jax.experimental import pallas as pl
from jax.experimental.pallas import tpu as pltpu
from jax.experimental.pallas import tpu_sc as plsc


def _msg_matmul(x, W_msg, nc, hh, ba):
    n, d = x.shape
    nb = n // ba

    def body(x_ref, w_ref, o_ref):
        o_ref[...] = jnp.dot(x_ref[...], w_ref[...],
                             preferred_element_type=jnp.float32)

    return pl.pallas_call(
        body,
        grid=(nc, nb),
        in_specs=[
            pl.BlockSpec((ba, d), lambda c, i: (i, 0)),
            pl.BlockSpec((d, hh), lambda c, i: (0, c)),
        ],
        out_specs=pl.BlockSpec((ba, hh), lambda c, i: (c * nb + i, 0)),
        out_shape=jax.ShapeDtypeStruct((nc * n, hh), jnp.float32),
    )(x, W_msg)


def _segment_sum_sc(src2, dst2, y2, zeros, n, hh, nc, ns, ch):
    npc = dst2.shape[0]
    cpt = npc // ns
    n_pad = zeros.shape[0] * ns
    rpt = zeros.shape[0]

    mesh = plsc.VectorSubcoreMesh(core_axis_name="c", subcore_axis_name="s")

    gblk = 16
    nblk = cpt // gblk
    assert cpt % gblk == 0

    @functools.partial(
        pl.kernel,
        out_type=jax.ShapeDtypeStruct((nc * n_pad, hh), jnp.float32),
        mesh=mesh,
        scratch_types=[
            pltpu.VMEM((cpt, ch), jnp.int32),
            pltpu.VMEM((2, gblk, ch), jnp.int32),
            pltpu.VMEM((2, ch, hh), jnp.float32),
            pltpu.VMEM_SHARED((n_pad, hh), jnp.float32),
            pltpu.SemaphoreType.DMA,
            pltpu.SemaphoreType.DMA,
            pltpu.SemaphoreType.DMA,
            pltpu.SemaphoreType.DMA,
            pltpu.SemaphoreType.DMA,
        ],
    )
    def segsum(src_hbm, dst_hbm, y_hbm, z_hbm, out_hbm, gi, si2, rows, acc,
               gs0, gs1, ss0, ss1, isem):
        c = lax.axis_index("c")
        s = lax.axis_index("s")
        gsem = (gs0, gs1)
        ssem = (ss0, ss1)
        pltpu.sync_copy(src_hbm.at[pl.ds(c * npc + s * cpt, cpt)], gi)
        dbase = s * cpt
        pltpu.sync_copy(dst_hbm.at[pl.ds(dbase, gblk)], si2.at[0])
        pltpu.async_copy(y_hbm.at[gi.at[0]], rows.at[0], gsem[0])
        pltpu.sync_copy(z_hbm, acc.at[pl.ds(s * rpt, rpt)])
        plsc.subcore_barrier()

        def body(jo, carry):
            sl = lax.rem(jo, 2)

            @pl.when(jo > 0)
            def _():
                pltpu.make_async_copy(
                    dst_hbm.at[pl.ds(dbase + jo * gblk, gblk)],
                    si2.at[sl], isem).wait()

            @pl.when(jo + 1 < nblk)
            def _():
                pltpu.async_copy(
                    dst_hbm.at[pl.ds(dbase + (jo + 1) * gblk, gblk)],
                    si2.at[1 - sl], isem)

            prev_scatter = [None]
            for b in range(gblk):
                j = jo * gblk + b
                buf = b % 2

                if prev_scatter[0] is not None:
                    prev_scatter[0].wait()
                    prev_scatter[0] = None

                @pl.when(j + 1 < cpt)
                def _():
                    pltpu.async_copy(y_hbm.at[gi.at[j + 1]],
                                     rows.at[1 - buf], gsem[1 - buf])

                pltpu.make_async_copy(y_hbm.at[gi.at[j]], rows.at[buf],
                                      gsem[buf]).wait()
                if b < gblk - 1:
                    prev_scatter[0] = pltpu.async_copy(
                        rows.at[buf], acc.at[si2.at[sl, b]],
                        ssem[buf], add=True)
                else:
                    pltpu.sync_copy(rows.at[buf], acc.at[si2.at[sl, b]],
                                    add=True)
            return carry

        lax.fori_loop(0, nblk, body, 0)
        plsc.subcore_barrier()
        pltpu.sync_copy(acc.at[pl.ds(s * rpt, rpt)],
                        out_hbm.at[pl.ds(c * n_pad + s * rpt, rpt)])

    return segsum(src2, dst2, y2, zeros)


def _pre_from_x(x, types2, U_cells, b_cells, bc):
    n, d = x.shape
    t1, _, h = U_cells.shape
    nb = n // bc

    def body(x_ref, t_ref, u_ref, b_ref, o_ref):
        xv = x_ref[...]
        tv = t_ref[...]
        out = jnp.zeros((bc, h), jnp.float32)
        for k in range(t1):
            pk = (jnp.dot(xv, u_ref[k], preferred_element_type=jnp.float32)
                  + b_ref[k])
            out = out + jnp.where(tv == k, pk, 0.0)
        o_ref[...] = out

    return pl.pallas_call(
        body,
        grid=(nb,),
        in_specs=[
            pl.BlockSpec((bc, d), lambda i: (i, 0)),
            pl.BlockSpec((bc, 1), lambda i: (i, 0)),
            pl.BlockSpec((t1, d, h), lambda i: (0, 0, 0)),
            pl.BlockSpec((t1, h), lambda i: (0, 0)),
        ],
        out_specs=pl.BlockSpec((bc, h), lambda i: (i, 0)),
        out_shape=jax.ShapeDtypeStruct((n, h), jnp.float32),
    )(x, types2, U_cells, b_cells)


def _apply_cells(agg2, q, types2, W_cells, hh, bc):
    n, h = q.shape
    t1 = W_cells.shape[0]
    nb = n // bc

    def body(agg_ref, q_ref, t_ref, w_ref, o_ref):
        a0 = agg_ref[0]
        a1 = agg_ref[1]
        tv = t_ref[...]
        acc = q_ref[...]
        for k in range(t1):
            wk = w_ref[k]
            pk = (jnp.dot(a0, wk[:hh, :], preferred_element_type=jnp.float32)
                  + jnp.dot(a1, wk[hh:, :], preferred_element_type=jnp.float32))
            acc = acc + jnp.where(tv == k, pk, 0.0)
        o_ref[...] = jnp.tanh(acc)

    return pl.pallas_call(
        body,
        grid=(nb,),
        in_specs=[
            pl.BlockSpec((2, bc, hh), lambda i: (0, i, 0)),
            pl.BlockSpec((bc, h), lambda i: (i, 0)),
            pl.BlockSpec((bc, 1), lambda i: (i, 0)),
            pl.BlockSpec((t1, h, h), lambda i: (0, 0, 0)),
        ],
        out_specs=pl.BlockSpec((bc, h), lambda i: (i, 0)),
        out_shape=jax.ShapeDtypeStruct((n, h), jnp.float32),
    )(agg2, q, types2, W_cells)


def kernel(x, edge_index, types, W_msg, W_cells, U_cells, b_cells):
    n, d = x.shape
    h = W_msg.shape[1]
    e = edge_index.shape[1]
    hh = h // 2
    nc, ns, ch = 2, 16, 128

    gran = ns * ch * 8
    e_pad = ((e + gran - 1) // gran) * gran
    pad = e_pad - e
    src = edge_index[0]
    dst = edge_index[1]
    n_pad = ((n + 1 + ns * 8 - 1) // (ns * 8)) * (ns * 8)
    pad_ar = jnp.arange(pad, dtype=jnp.int32)
    src_p = jnp.concatenate([src, (pad_ar * 97) % n])
    dst_p = jnp.concatenate([dst, n + pad_ar % (n_pad - n)])
    src2 = jnp.stack([src_p, src_p + n]).reshape(nc * (e_pad // ch), ch)
    dst2 = dst_p.reshape(e_pad // ch, ch)
    zeros = jnp.zeros((n_pad // ns, hh), jnp.float32)

    types2 = types.reshape(n, 1)
    y2 = _msg_matmul(x, W_msg, nc, hh, ba=1000)
    agg_flat = _segment_sum_sc(src2, dst2, y2, zeros, n, hh, nc, ns, ch)
    q = _pre_from_x(x, types2, U_cells, b_cells, bc=400)
    agg2 = agg_flat.reshape(nc, n_pad, hh)
    out = _apply_cells(agg2, q, types2, W_cells, hh, bc=400)
    return out


# speedup vs baseline: 8.0569x; 1.1295x over previous
import functools

import jax
import jax.numpy as jnp
from jax import lax
from jax.experimental import pallas as pl
from jax.experimental.pallas import tpu as pltpu
from jax.experimental.pallas import tpu_sc as plsc


def _msg_matmul(x, W_msg, nc, hh, ba):
    n, d = x.shape
    nb = n // ba

    def body(x_ref, w_ref, o_ref):
        y = jnp.dot(x_ref[...], w_ref[...],
                    preferred_element_type=jnp.float32)
        for c in range(nc):
            o_ref[c] = y[:, c * hh:(c + 1) * hh]

    return pl.pallas_call(
        body,
        grid=(nb,),
        in_specs=[
            pl.BlockSpec((ba, d), lambda i: (i, 0)),
            pl.BlockSpec((d, nc * hh), lambda i: (0, 0)),
        ],
        out_specs=pl.BlockSpec((nc, ba, hh), lambda i: (0, i, 0)),
        out_shape=jax.ShapeDtypeStruct((nc, n, hh), jnp.float32),
    )(x, W_msg)


def _segment_sum_sc(src2, dst2, y2, zeros, n, hh, nc, ns, ch):
    npc = dst2.shape[0]
    cpt = npc // ns
    n_pad = zeros.shape[0] * ns
    rpt = zeros.shape[0]

    mesh = plsc.VectorSubcoreMesh(core_axis_name="c", subcore_axis_name="s")

    gblk = 16
    nblk = cpt // gblk
    assert cpt % gblk == 0

    @functools.partial(
        pl.kernel,
        out_type=jax.ShapeDtypeStruct((nc, n_pad, hh), jnp.float32),
        mesh=mesh,
        scratch_types=[
            pltpu.VMEM((cpt, ch), jnp.int32),
            pltpu.VMEM((2, gblk, ch), jnp.int32),
            pltpu.VMEM((2, ch, hh), jnp.float32),
            pltpu.VMEM_SHARED((n_pad, hh), jnp.float32),
            pltpu.SemaphoreType.DMA,
            pltpu.SemaphoreType.DMA,
            pltpu.SemaphoreType.DMA,
            pltpu.SemaphoreType.DMA,
            pltpu.SemaphoreType.DMA,
        ],
    )
    def segsum(src_hbm, dst_hbm, y_hbm, z_hbm, out_hbm, gi, si2, rows, acc,
               gs0, gs1, ss0, ss1, isem):
        c = lax.axis_index("c")
        s = lax.axis_index("s")
        gsem = (gs0, gs1)
        ssem = (ss0, ss1)
        tbl = y_hbm.at[c]
        dbase = s * cpt
        pltpu.sync_copy(src_hbm.at[pl.ds(dbase, cpt)], gi)
        pltpu.sync_copy(dst_hbm.at[pl.ds(dbase, gblk)], si2.at[0])
        pltpu.async_copy(tbl.at[gi.at[0]], rows.at[0], gsem[0])
        pltpu.sync_copy(z_hbm, acc.at[pl.ds(s * rpt, rpt)])
        plsc.subcore_barrier()

        def body(jo, carry):
            sl = lax.rem(jo, 2)

            @pl.when(jo > 0)
            def _():
                pltpu.make_async_copy(
                    dst_hbm.at[pl.ds(dbase + jo * gblk, gblk)],
                    si2.at[sl], isem).wait()

            @pl.when(jo + 1 < nblk)
            def _():
                pltpu.async_copy(
                    dst_hbm.at[pl.ds(dbase + (jo + 1) * gblk, gblk)],
                    si2.at[1 - sl], isem)

            prev_scatter = [None]
            for b in range(gblk):
                j = jo * gblk + b
                buf = b % 2

                if prev_scatter[0] is not None:
                    prev_scatter[0].wait()
                    prev_scatter[0] = None

                @pl.when(j + 1 < cpt)
                def _():
                    pltpu.async_copy(tbl.at[gi.at[j + 1]],
                                     rows.at[1 - buf], gsem[1 - buf])

                pltpu.make_async_copy(tbl.at[gi.at[j]], rows.at[buf],
                                      gsem[buf]).wait()
                if b < gblk - 1:
                    prev_scatter[0] = pltpu.async_copy(
                        rows.at[buf], acc.at[si2.at[sl, b]],
                        ssem[buf], add=True)
                else:
                    pltpu.sync_copy(rows.at[buf], acc.at[si2.at[sl, b]],
                                    add=True)
            return carry

        lax.fori_loop(0, nblk, body, 0)
        plsc.subcore_barrier()
        pltpu.sync_copy(acc.at[pl.ds(s * rpt, rpt)],
                        out_hbm.at[c].at[pl.ds(s * rpt, rpt)])

    return segsum(src2, dst2, y2, zeros)


def _pre_from_x(x, types2, U_cells, b_cells, bc):
    n, d = x.shape
    t1, _, h = U_cells.shape
    nb = n // bc

    def body(x_ref, t_ref, u_ref, b_ref, o_ref):
        xv = x_ref[...]
        tv = t_ref[...]
        out = jnp.zeros((bc, h), jnp.float32)
        for k in range(t1):
            pk = (jnp.dot(xv, u_ref[k], preferred_element_type=jnp.float32)
                  + b_ref[k])
            out = out + jnp.where(tv == k, pk, 0.0)
        o_ref[...] = out

    return pl.pallas_call(
        body,
        grid=(nb,),
        in_specs=[
            pl.BlockSpec((bc, d), lambda i: (i, 0)),
            pl.BlockSpec((bc, 1), lambda i: (i, 0)),
            pl.BlockSpec((t1, d, h), lambda i: (0, 0, 0)),
            pl.BlockSpec((t1, h), lambda i: (0, 0)),
        ],
        out_specs=pl.BlockSpec((bc, h), lambda i: (i, 0)),
        out_shape=jax.ShapeDtypeStruct((n, h), jnp.float32),
    )(x, types2, U_cells, b_cells)


def _apply_cells(agg2, q, types2, W_cells, hh, bc):
    n, h = q.shape
    t1 = W_cells.shape[0]
    nb = n // bc

    def body(agg_ref, q_ref, t_ref, w_ref, o_ref):
        a0 = agg_ref[0]
        a1 = agg_ref[1]
        tv = t_ref[...]
        acc = q_ref[...]
        for k in range(t1):
            wk = w_ref[k]
            pk = (jnp.dot(a0, wk[:hh, :], preferred_element_type=jnp.float32)
                  + jnp.dot(a1, wk[hh:, :], preferred_element_type=jnp.float32))
            acc = acc + jnp.where(tv == k, pk, 0.0)
        o_ref[...] = jnp.tanh(acc)

    return pl.pallas_call(
        body,
        grid=(nb,),
        in_specs=[
            pl.BlockSpec((2, bc, hh), lambda i: (0, i, 0)),
            pl.BlockSpec((bc, h), lambda i: (i, 0)),
            pl.BlockSpec((bc, 1), lambda i: (i, 0)),
            pl.BlockSpec((t1, h, h), lambda i: (0, 0, 0)),
        ],
        out_specs=pl.BlockSpec((bc, h), lambda i: (i, 0)),
        out_shape=jax.ShapeDtypeStruct((n, h), jnp.float32),
    )(agg2, q, types2, W_cells)


def kernel(x, edge_index, types, W_msg, W_cells, U_cells, b_cells):
    n, d = x.shape
    h = W_msg.shape[1]
    e = edge_index.shape[1]
    hh = h // 2
    nc, ns, ch = 2, 16, 128

    gran = ns * ch * 8
    e_pad = ((e + gran - 1) // gran) * gran
    pad = e_pad - e
    src = edge_index[0]
    dst = edge_index[1]
    n_pad = ((n + 1 + ns * 8 - 1) // (ns * 8)) * (ns * 8)
    pad_ar = jnp.arange(pad, dtype=jnp.int32)
    src_p = jnp.concatenate([src, (pad_ar * 97) % n])
    dst_p = jnp.concatenate([dst, n + pad_ar % (n_pad - n)])
    src2 = src_p.reshape(e_pad // ch, ch)
    dst2 = dst_p.reshape(e_pad // ch, ch)
    zeros = jnp.zeros((n_pad // ns, hh), jnp.float32)

    types2 = types.reshape(n, 1)
    y2 = _msg_matmul(x, W_msg, nc, hh, ba=1000)
    agg2 = _segment_sum_sc(src2, dst2, y2, zeros, n, hh, nc, ns, ch)
    q = _pre_from_x(x, types2, U_cells, b_cells, bc=400)
    out = _apply_cells(agg2, q, types2, W_cells, hh, bc=1000)
    return out


# speedup vs baseline: 8.2417x; 1.0229x over previous
import functools

import jax
import jax.numpy as jnp
from jax import lax
from jax.experimental import pallas as pl
from jax.experimental.pallas import tpu as pltpu
from jax.experimental.pallas import tpu_sc as plsc


def _msg_matmul(x, W_msg, nc, hh, ba):
    n, d = x.shape
    nb = n // ba

    def body(x_ref, w_ref, o_ref):
        y = jnp.dot(x_ref[...], w_ref[...],
                    preferred_element_type=jnp.float32)
        for c in range(nc):
            o_ref[c] = y[:, c * hh:(c + 1) * hh]

    return pl.pallas_call(
        body,
        grid=(nb,),
        in_specs=[
            pl.BlockSpec((ba, d), lambda i: (i, 0)),
            pl.BlockSpec((d, nc * hh), lambda i: (0, 0)),
        ],
        out_specs=pl.BlockSpec((nc, ba, hh), lambda i: (0, i, 0)),
        out_shape=jax.ShapeDtypeStruct((nc, n, hh), jnp.float32),
    )(x, W_msg)


def _segment_sum_sc(src2, dst2, y2, zeros, n, hh, nc, ns, ch):
    npc = dst2.shape[0]
    cpt = npc // ns
    n_pad = zeros.shape[0] * ns
    rpt = zeros.shape[0]

    mesh = plsc.VectorSubcoreMesh(core_axis_name="c", subcore_axis_name="s")

    gblk = 16
    nblk = cpt // gblk
    assert cpt % gblk == 0

    @functools.partial(
        pl.kernel,
        out_type=jax.ShapeDtypeStruct((nc, n_pad, hh), jnp.float32),
        mesh=mesh,
        scratch_types=[
            pltpu.VMEM((cpt, ch), jnp.int32),
            pltpu.VMEM((2, gblk, ch), jnp.int32),
            pltpu.VMEM((2, ch, hh), jnp.float32),
            pltpu.VMEM_SHARED((n_pad, hh), jnp.float32),
            pltpu.SemaphoreType.DMA,
            pltpu.SemaphoreType.DMA,
            pltpu.SemaphoreType.DMA,
            pltpu.SemaphoreType.DMA,
            pltpu.SemaphoreType.DMA,
        ],
    )
    def segsum(src_hbm, dst_hbm, y_hbm, z_hbm, out_hbm, gi, si2, rows, acc,
               gs0, gs1, ss0, ss1, isem):
        c = lax.axis_index("c")
        s = lax.axis_index("s")
        gsem = (gs0, gs1)
        ssem = (ss0, ss1)
        tbl = y_hbm.at[c]
        dbase = s * cpt
        pltpu.sync_copy(src_hbm.at[pl.ds(dbase, cpt)], gi)
        pltpu.sync_copy(dst_hbm.at[pl.ds(dbase, gblk)], si2.at[0])
        pltpu.async_copy(tbl.at[gi.at[0]], rows.at[0], gsem[0])
        pltpu.sync_copy(z_hbm, acc.at[pl.ds(s * rpt, rpt)])
        plsc.subcore_barrier()

        def body(jo, carry):
            sl = lax.rem(jo, 2)

            @pl.when(jo > 0)
            def _():
                pltpu.make_async_copy(
                    dst_hbm.at[pl.ds(dbase + jo * gblk, gblk)],
                    si2.at[sl], isem).wait()

            @pl.when(jo + 1 < nblk)
            def _():
                pltpu.async_copy(
                    dst_hbm.at[pl.ds(dbase + (jo + 1) * gblk, gblk)],
                    si2.at[1 - sl], isem)

            prev_scatter = [None]
            for b in range(gblk):
                j = jo * gblk + b
                buf = b % 2

                if prev_scatter[0] is not None:
                    prev_scatter[0].wait()
                    prev_scatter[0] = None

                @pl.when(j + 1 < cpt)
                def _():
                    pltpu.async_copy(tbl.at[gi.at[j + 1]],
                                     rows.at[1 - buf], gsem[1 - buf])

                pltpu.make_async_copy(tbl.at[gi.at[j]], rows.at[buf],
                                      gsem[buf]).wait()
                if b < gblk - 1:
                    prev_scatter[0] = pltpu.async_copy(
                        rows.at[buf], acc.at[si2.at[sl, b]],
                        ssem[buf], add=True)
                else:
                    pltpu.sync_copy(rows.at[buf], acc.at[si2.at[sl, b]],
                                    add=True)
            return carry

        lax.fori_loop(0, nblk, body, 0)
        plsc.subcore_barrier()
        pltpu.sync_copy(acc.at[pl.ds(s * rpt, rpt)],
                        out_hbm.at[c].at[pl.ds(s * rpt, rpt)])

    return segsum(src2, dst2, y2, zeros)


def _pre_from_x(x, types2, U_cells, b_cells, bc):
    n, d = x.shape
    t1, _, h = U_cells.shape
    nb = n // bc

    def body(x_ref, t_ref, u_ref, b_ref, o_ref):
        xv = x_ref[...]
        tv = t_ref[...]
        out = jnp.zeros((bc, h), jnp.float32)
        for k in range(t1):
            pk = (jnp.dot(xv, u_ref[k], preferred_element_type=jnp.float32)
                  + b_ref[k])
            out = out + jnp.where(tv == k, pk, 0.0)
        o_ref[...] = out

    return pl.pallas_call(
        body,
        grid=(nb,),
        in_specs=[
            pl.BlockSpec((bc, d), lambda i: (i, 0)),
            pl.BlockSpec((bc, 1), lambda i: (i, 0)),
            pl.BlockSpec((t1, d, h), lambda i: (0, 0, 0)),
            pl.BlockSpec((t1, h), lambda i: (0, 0)),
        ],
        out_specs=pl.BlockSpec((bc, h), lambda i: (i, 0)),
        out_shape=jax.ShapeDtypeStruct((n, h), jnp.float32),
    )(x, types2, U_cells, b_cells)


def _apply_cells(agg2, q, types2, W_cells, hh, bc):
    n, h = q.shape
    t1 = W_cells.shape[0]
    nb = n // bc

    def body(agg_ref, q_ref, t_ref, w_ref, o_ref):
        a0 = agg_ref[0]
        a1 = agg_ref[1]
        tv = t_ref[...]
        acc = q_ref[...]
        for k in range(t1):
            wk = w_ref[k]
            pk = (jnp.dot(a0, wk[:hh, :], preferred_element_type=jnp.float32)
                  + jnp.dot(a1, wk[hh:, :], preferred_element_type=jnp.float32))
            acc = acc + jnp.where(tv == k, pk, 0.0)
        o_ref[...] = jnp.tanh(acc)

    return pl.pallas_call(
        body,
        grid=(nb,),
        in_specs=[
            pl.BlockSpec((2, bc, hh), lambda i: (0, i, 0)),
            pl.BlockSpec((bc, h), lambda i: (i, 0)),
            pl.BlockSpec((bc, 1), lambda i: (i, 0)),
            pl.BlockSpec((t1, h, h), lambda i: (0, 0, 0)),
        ],
        out_specs=pl.BlockSpec((bc, h), lambda i: (i, 0)),
        out_shape=jax.ShapeDtypeStruct((n, h), jnp.float32),
    )(agg2, q, types2, W_cells)


def kernel(x, edge_index, types, W_msg, W_cells, U_cells, b_cells):
    n, d = x.shape
    h = W_msg.shape[1]
    e = edge_index.shape[1]
    hh = h // 2
    nc, ns, ch = 2, 16, 128

    gran = ns * ch * 8
    e_pad = ((e + gran - 1) // gran) * gran
    pad = e_pad - e
    src = edge_index[0]
    dst = edge_index[1]
    n_pad = ((n + 1 + ns * 8 - 1) // (ns * 8)) * (ns * 8)
    pad_ar = jnp.arange(pad, dtype=jnp.int32)
    src_p = jnp.concatenate([src, (pad_ar * 97) % n])
    dst_p = jnp.concatenate([dst, n + pad_ar % (n_pad - n)])
    src2 = src_p.reshape(e_pad // ch, ch)
    dst2 = dst_p.reshape(e_pad // ch, ch)
    zeros = jnp.zeros((n_pad // ns, hh), jnp.float32)

    types2 = types.reshape(n, 1)
    y2 = _msg_matmul(x, W_msg, nc, hh, ba=2000)
    agg2 = _segment_sum_sc(src2, dst2, y2, zeros, n, hh, nc, ns, ch)
    q = _pre_from_x(x, types2, U_cells, b_cells, bc=400)
    out = _apply_cells(agg2, q, types2, W_cells, hh, bc=2000)
    return out


# speedup vs baseline: 8.2474x; 1.0007x over previous
import functools

import jax
import jax.numpy as jnp
from jax import lax
from jax.experimental import pallas as pl
from jax.experimental.pallas import tpu as pltpu
from jax.experimental.pallas import tpu_sc as plsc


def _msg_matmul(x, W_msg, nc, hh, ba):
    n, d = x.shape
    nb = n // ba

    def body(x_ref, w_ref, o_ref):
        y = jnp.dot(x_ref[...], w_ref[...],
                    preferred_element_type=jnp.float32)
        for c in range(nc):
            o_ref[c] = y[:, c * hh:(c + 1) * hh]

    return pl.pallas_call(
        body,
        grid=(nb,),
        in_specs=[
            pl.BlockSpec((ba, d), lambda i: (i, 0)),
            pl.BlockSpec((d, nc * hh), lambda i: (0, 0)),
        ],
        out_specs=pl.BlockSpec((nc, ba, hh), lambda i: (0, i, 0)),
        out_shape=jax.ShapeDtypeStruct((nc, n, hh), jnp.float32),
    )(x, W_msg)


def _edge_indices(edge3, e, n, n_pad, e_pad, ch):
    npc = e_pad // ch
    nb = 10
    rb = npc // nb
    ndum = n_pad - n

    def body(e_ref, s_ref, d_ref):
        i = pl.program_id(0)
        pos = (i * (rb * ch)
               + lax.broadcasted_iota(jnp.int32, (rb, ch), 0) * ch
               + lax.broadcasted_iota(jnp.int32, (rb, ch), 1))
        valid = pos < e
        s_ref[...] = jnp.where(valid, e_ref[0], (pos * 97) % n)
        d_ref[...] = jnp.where(valid, e_ref[1], n + pos % ndum)

    return pl.pallas_call(
        body,
        grid=(nb,),
        in_specs=[pl.BlockSpec((2, rb, ch), lambda i: (0, i, 0))],
        out_specs=[pl.BlockSpec((rb, ch), lambda i: (i, 0)),
                   pl.BlockSpec((rb, ch), lambda i: (i, 0))],
        out_shape=[jax.ShapeDtypeStruct((npc, ch), jnp.int32),
                   jax.ShapeDtypeStruct((npc, ch), jnp.int32)],
    )(edge3)


def _segment_sum_sc(src2, dst2, y2, zeros, n, hh, nc, ns, ch):
    npc = dst2.shape[0]
    cpt = npc // ns
    n_pad = zeros.shape[0] * ns
    rpt = zeros.shape[0]

    mesh = plsc.VectorSubcoreMesh(core_axis_name="c", subcore_axis_name="s")

    gblk = 16
    nblk = cpt // gblk
    assert cpt % gblk == 0

    @functools.partial(
        pl.kernel,
        out_type=jax.ShapeDtypeStruct((nc, n_pad, hh), jnp.float32),
        mesh=mesh,
        scratch_types=[
            pltpu.VMEM((cpt, ch), jnp.int32),
            pltpu.VMEM((2, gblk, ch), jnp.int32),
            pltpu.VMEM((2, ch, hh), jnp.float32),
            pltpu.VMEM_SHARED((n_pad, hh), jnp.float32),
            pltpu.SemaphoreType.DMA,
            pltpu.SemaphoreType.DMA,
            pltpu.SemaphoreType.DMA,
            pltpu.SemaphoreType.DMA,
            pltpu.SemaphoreType.DMA,
        ],
    )
    def segsum(src_hbm, dst_hbm, y_hbm, z_hbm, out_hbm, gi, si2, rows, acc,
               gs0, gs1, ss0, ss1, isem):
        c = lax.axis_index("c")
        s = lax.axis_index("s")
        gsem = (gs0, gs1)
        ssem = (ss0, ss1)
        tbl = y_hbm.at[c]
        dbase = s * cpt
        pltpu.sync_copy(src_hbm.at[pl.ds(dbase, cpt)], gi)
        pltpu.sync_copy(dst_hbm.at[pl.ds(dbase, gblk)], si2.at[0])
        pltpu.async_copy(tbl.at[gi.at[0]], rows.at[0], gsem[0])
        pltpu.sync_copy(z_hbm, acc.at[pl.ds(s * rpt, rpt)])
        plsc.subcore_barrier()

        def body(jo, carry):
            sl = lax.rem(jo, 2)

            @pl.when(jo > 0)
            def _():
                pltpu.make_async_copy(
                    dst_hbm.at[pl.ds(dbase + jo * gblk, gblk)],
                    si2.at[sl], isem).wait()

            @pl.when(jo + 1 < nblk)
            def _():
                pltpu.async_copy(
                    dst_hbm.at[pl.ds(dbase + (jo + 1) * gblk, gblk)],
                    si2.at[1 - sl], isem)

            prev_scatter = [None]
            for b in range(gblk):
                j = jo * gblk + b
                buf = b % 2

                if prev_scatter[0] is not None:
                    prev_scatter[0].wait()
                    prev_scatter[0] = None

                @pl.when(j + 1 < cpt)
                def _():
                    pltpu.async_copy(tbl.at[gi.at[j + 1]],
                                     rows.at[1 - buf], gsem[1 - buf])

                pltpu.make_async_copy(tbl.at[gi.at[j]], rows.at[buf],
                                      gsem[buf]).wait()
                if b < gblk - 1:
                    prev_scatter[0] = pltpu.async_copy(
                        rows.at[buf], acc.at[si2.at[sl, b]],
                        ssem[buf], add=True)
                else:
                    pltpu.sync_copy(rows.at[buf], acc.at[si2.at[sl, b]],
                                    add=True)
            return carry

        lax.fori_loop(0, nblk, body, 0)
        plsc.subcore_barrier()
        pltpu.sync_copy(acc.at[pl.ds(s * rpt, rpt)],
                        out_hbm.at[c].at[pl.ds(s * rpt, rpt)])

    return segsum(src2, dst2, y2, zeros)


def _pre_from_x(x, types2, U_cells, b_cells, bc):
    n, d = x.shape
    t1, _, h = U_cells.shape
    nb = n // bc

    def body(x_ref, t_ref, u_ref, b_ref, o_ref):
        xv = x_ref[...]
        tv = t_ref[...]
        out = jnp.zeros((bc, h), jnp.float32)
        for k in range(t1):
            pk = (jnp.dot(xv, u_ref[k], preferred_element_type=jnp.float32)
                  + b_ref[k])
            out = out + jnp.where(tv == k, pk, 0.0)
        o_ref[...] = out

    return pl.pallas_call(
        body,
        grid=(nb,),
        in_specs=[
            pl.BlockSpec((bc, d), lambda i: (i, 0)),
            pl.BlockSpec((bc, 1), lambda i: (i, 0)),
            pl.BlockSpec((t1, d, h), lambda i: (0, 0, 0)),
            pl.BlockSpec((t1, h), lambda i: (0, 0)),
        ],
        out_specs=pl.BlockSpec((bc, h), lambda i: (i, 0)),
        out_shape=jax.ShapeDtypeStruct((n, h), jnp.float32),
    )(x, types2, U_cells, b_cells)


def _apply_cells(agg2, q, types2, W_cells, hh, bc):
    n, h = q.shape
    t1 = W_cells.shape[0]
    nb = n // bc

    def body(agg_ref, q_ref, t_ref, w_ref, o_ref):
        a0 = agg_ref[0]
        a1 = agg_ref[1]
        tv = t_ref[...]
        acc = q_ref[...]
        for k in range(t1):
            wk = w_ref[k]
            pk = (jnp.dot(a0, wk[:hh, :], preferred_element_type=jnp.float32)
                  + jnp.dot(a1, wk[hh:, :], preferred_element_type=jnp.float32))
            acc = acc + jnp.where(tv == k, pk, 0.0)
        o_ref[...] = jnp.tanh(acc)

    return pl.pallas_call(
        body,
        grid=(nb,),
        in_specs=[
            pl.BlockSpec((2, bc, hh), lambda i: (0, i, 0)),
            pl.BlockSpec((bc, h), lambda i: (i, 0)),
            pl.BlockSpec((bc, 1), lambda i: (i, 0)),
            pl.BlockSpec((t1, h, h), lambda i: (0, 0, 0)),
        ],
        out_specs=pl.BlockSpec((bc, h), lambda i: (i, 0)),
        out_shape=jax.ShapeDtypeStruct((n, h), jnp.float32),
    )(agg2, q, types2, W_cells)


def kernel(x, edge_index, types, W_msg, W_cells, U_cells, b_cells):
    n, d = x.shape
    h = W_msg.shape[1]
    e = edge_index.shape[1]
    hh = h // 2
    nc, ns, ch = 2, 16, 128

    gran = ns * ch * 8
    e_pad = ((e + gran - 1) // gran) * gran
    n_pad = ((n + 1 + ns * 8 - 1) // (ns * 8)) * (ns * 8)
    src2, dst2 = _edge_indices(edge_index.reshape(2, e // ch, ch),
                               e, n, n_pad, e_pad, ch)
    zeros = jnp.zeros((n_pad // ns, hh), jnp.float32)

    types2 = types.reshape(n, 1)
    y2 = _msg_matmul(x, W_msg, nc, hh, ba=2000)
    agg2 = _segment_sum_sc(src2, dst2, y2, zeros, n, hh, nc, ns, ch)
    q = _pre_from_x(x, types2, U_cells, b_cells, bc=400)
    out = _apply_cells(agg2, q, types2, W_cells, hh, bc=2000)
    return out
